# baseline (speedup 1.0000x reference)
import jax
import jax.numpy as jnp
from jax import lax
from jax.experimental import pallas as pl
from jax.experimental.pallas import tpu as pltpu
from jax.experimental.pallas import tpu_sc as plsc

NC = 2
NS = 16
NW = NC * NS
LANES = 16
G = 64


def _fill_f32(ref, rows, cols, val):
    zv = jnp.full((LANES,), val, jnp.float32)

    def bi(i, carry):
        def bj(j, c):
            ref[i, pl.ds(j * LANES, LANES)] = zv
            return c

        return lax.fori_loop(0, cols // LANES, bj, carry)

    lax.fori_loop(0, rows, bi, 0)


def _strided_chunks(s, nzch, fn):
    def step(i, carry):
        k = s + i * NS

        @pl.when(k < nzch)
        def _():
            fn(k)

        return carry

    lax.fori_loop(0, (nzch + NS - 1) // NS, step, 0)


def _make_agg(N, H, K, NCH, deg_too=False):
    assert N % K == 0 and K == 80
    nzch = N // K
    mesh = plsc.VectorSubcoreMesh(core_axis_name="c", subcore_axis_name="s")
    out_type = [jax.ShapeDtypeStruct((NC, N, H), jnp.float32)]
    if deg_too:
        out_type.append(jax.ShapeDtypeStruct((NC, N, H), jnp.float32))

    def body(*refs):
        if deg_too:
            (h_hbm, src_hbm, dst_hbm, part_hbm, degp_hbm, src_v, dst_v,
             rows0, acc_sh, rows1, sem0, sem1, sems0, sems1) = refs
        else:
            (h_hbm, src_hbm, dst_hbm, part_hbm, src_v, dst_v, rows0,
             acc_sh, rows1, sem0, sem1, sems0, sems1) = refs
        c = lax.axis_index("c")
        s = lax.axis_index("s")
        w = c * NS + s

        a_src = pltpu.async_copy(src_hbm.at[w], src_v, sem0)
        a_dst = pltpu.async_copy(dst_hbm.at[w], dst_v, sem1)

        def zero_acc():
            _strided_chunks(
                s, nzch,
                lambda k: pltpu.sync_copy(rows0, acc_sh.at[pl.ds(k * K, K)]))

        _fill_f32(rows0, K, H, 0.0)
        zero_acc()

        if deg_too:
            _fill_f32(rows1, K, H, 1.0)
            a_src.wait()
            a_dst.wait()
            plsc.subcore_barrier()

            def dchunk(j, carry):
                pltpu.sync_copy(rows1, acc_sh.at[dst_v.at[j]], add=True)
                return carry

            lax.fori_loop(0, NCH, dchunk, 0)
            plsc.subcore_barrier()
            _strided_chunks(
                s, nzch,
                lambda k: pltpu.sync_copy(acc_sh.at[pl.ds(k * K, K)],
                                          degp_hbm.at[c].at[pl.ds(k * K, K)]))
            zero_acc()
        else:
            a_src.wait()
            a_dst.wait()
        plsc.subcore_barrier()

        def gath(j, buf, sem):
            for off, ln in ((0, 24), (24, 24), (48, 16), (64, 16)):
                pltpu.async_copy(h_hbm.at[src_v.at[pl.ds(j * K + off, ln)]],
                                 buf.at[pl.ds(off, ln)], sem)

        def gwait(buf, sem):
            pltpu.make_async_copy(h_hbm.at[pl.ds(0, K)], buf, sem).wait()

        def sca(j, buf, sem):
            pltpu.async_copy(buf, acc_sh.at[dst_v.at[j]], sem, add=True)

        def swait(buf, sem):
            pltpu.make_async_copy(buf, acc_sh.at[pl.ds(0, K)], sem).wait()

        gath(0, rows0, sem0)
        gath(1, rows1, sem1)

        def dbody(t, carry):
            jj = 2 * t
            gwait(rows0, sem0)
            sca(jj, rows0, sems0)
            gwait(rows1, sem1)
            sca(jj + 1, rows1, sems1)
            swait(rows0, sems0)

            @pl.when(jj + 2 < NCH)
            def _():
                gath(jj + 2, rows0, sem0)

            swait(rows1, sems1)

            @pl.when(jj + 3 < NCH)
            def _():
                gath(jj + 3, rows1, sem1)

            return carry

        lax.fori_loop(0, NCH // 2, dbody, 0)
        if NCH % 2 == 1:
            gwait(rows0, sem0)
            pltpu.sync_copy(rows0, acc_sh.at[dst_v.at[NCH - 1]], add=True)
        plsc.subcore_barrier()

        _strided_chunks(
            s, nzch,
            lambda k: pltpu.sync_copy(acc_sh.at[pl.ds(k * K, K)],
                                      part_hbm.at[c].at[pl.ds(k * K, K)]))

    return pl.kernel(
        body,
        out_type=out_type,
        mesh=mesh,
        scratch_types=[
            pltpu.VMEM((NCH * K,), jnp.int32),
            pltpu.VMEM((NCH, K), jnp.int32),
            pltpu.VMEM((K, H), jnp.float32),
            pltpu.VMEM_SHARED((N, H), jnp.float32),
            pltpu.VMEM((K, H), jnp.float32),
            pltpu.SemaphoreType.DMA,
            pltpu.SemaphoreType.DMA,
            pltpu.SemaphoreType.DMA,
            pltpu.SemaphoreType.DMA,
        ])


def _make_deghist(NBLK, BE, NA):
    def body(d_ref, o_ref, acc):
        i = pl.program_id(0)

        @pl.when(i == 0)
        def _():
            acc[...] = jnp.zeros_like(acc)

        d = d_ref[0, 0, :]
        a = lax.shift_right_logical(d, 7)
        b = jnp.bitwise_and(d, 127)
        oa = (lax.broadcasted_iota(jnp.int32, (NA, BE), 0)
              == a[None, :]).astype(jnp.bfloat16)
        ob = (lax.broadcasted_iota(jnp.int32, (BE, 128), 1)
              == b[:, None]).astype(jnp.bfloat16)
        acc[...] += jnp.dot(oa, ob, preferred_element_type=jnp.float32)

        @pl.when(i == NBLK - 1)
        def _():
            o_ref[...] = acc[...]

    return pl.pallas_call(
        body,
        grid=(NBLK,),
        in_specs=[pl.BlockSpec((1, 1, BE), lambda i: (i, 0, 0))],
        out_specs=pl.BlockSpec((NA, 128), lambda i: (0, 0)),
        out_shape=jax.ShapeDtypeStruct((NA, 128), jnp.float32),
        scratch_shapes=[pltpu.VMEM((NA, 128), jnp.float32)],
    )


def _make_update0(N, H, BN):
    grid = (N // BN,)

    def body(part_ref, deg_ref, h_ref, wl_ref, wr_ref, b_ref, o_ref,
             rdeg_ref):
        rdeg = 1.0 / jnp.maximum(deg_ref[...], 1.0)
        rdeg_ref[...] = jnp.broadcast_to(rdeg, rdeg_ref.shape)
        agg = (part_ref[0] + part_ref[1]) * rdeg
        acc = jnp.dot(agg, wl_ref[...], preferred_element_type=jnp.float32)
        acc = acc + jnp.dot(h_ref[...], wr_ref[...],
                            preferred_element_type=jnp.float32)
        o_ref[...] = jnp.maximum(acc + b_ref[...], 0.0)

    return pl.pallas_call(
        body,
        grid=grid,
        in_specs=[
            pl.BlockSpec((NC, BN, H), lambda i: (0, i, 0)),
            pl.BlockSpec((BN, 1), lambda i: (i, 0)),
            pl.BlockSpec((BN, H), lambda i: (i, 0)),
            pl.BlockSpec((H, H), lambda i: (0, 0)),
            pl.BlockSpec((H, H), lambda i: (0, 0)),
            pl.BlockSpec((1, H), lambda i: (0, 0)),
        ],
        out_specs=[
            pl.BlockSpec((BN, H), lambda i: (i, 0)),
            pl.BlockSpec((BN, H), lambda i: (i, 0)),
        ],
        out_shape=[
            jax.ShapeDtypeStruct((N, H), jnp.float32),
            jax.ShapeDtypeStruct((N, H), jnp.float32),
        ],
    )


def _make_update(N, H, BN):
    grid = (N // BN,)

    def body(part_ref, rdeg_ref, h_ref, wl_ref, wr_ref, b_ref, o_ref):
        agg = (part_ref[0] + part_ref[1]) * rdeg_ref[:, :1]
        acc = jnp.dot(agg, wl_ref[...], preferred_element_type=jnp.float32)
        acc = acc + jnp.dot(h_ref[...], wr_ref[...],
                            preferred_element_type=jnp.float32)
        o_ref[...] = jnp.maximum(acc + b_ref[...], 0.0)

    return pl.pallas_call(
        body,
        grid=grid,
        in_specs=[
            pl.BlockSpec((NC, BN, H), lambda i: (0, i, 0)),
            pl.BlockSpec((BN, H), lambda i: (i, 0)),
            pl.BlockSpec((BN, H), lambda i: (i, 0)),
            pl.BlockSpec((H, H), lambda i: (0, 0)),
            pl.BlockSpec((H, H), lambda i: (0, 0)),
            pl.BlockSpec((1, H), lambda i: (0, 0)),
        ],
        out_specs=pl.BlockSpec((BN, H), lambda i: (i, 0)),
        out_shape=jax.ShapeDtypeStruct((N, H), jnp.float32),
    )


def _make_update_pool(N, H, C, BN):
    nb = N // BN

    def body(part_ref, rdeg_ref, h_ref, wl_ref, wr_ref, b_ref, bt_ref,
             w1_ref, b1_ref, w2_ref, b2_ref, o_ref, sums, cnts):
        i = pl.program_id(0)

        @pl.when(i == 0)
        def _():
            sums[...] = jnp.zeros_like(sums)
            cnts[...] = jnp.zeros_like(cnts)

        agg = (part_ref[0] + part_ref[1]) * rdeg_ref[:, :1]
        acc = jnp.dot(agg, wl_ref[...], preferred_element_type=jnp.float32)
        acc = acc + jnp.dot(h_ref[...], wr_ref[...],
                            preferred_element_type=jnp.float32)
        h3 = jnp.maximum(acc + b_ref[...], 0.0)

        bt = bt_ref[...][:, 0]
        onehot = (lax.broadcasted_iota(jnp.int32, (G, BN), 0)
                  == bt[None, :]).astype(jnp.float32)
        sums[...] += jnp.dot(onehot, h3, preferred_element_type=jnp.float32)
        cnts[...] += jnp.sum(onehot, axis=1, keepdims=True)

        @pl.when(i == nb - 1)
        def _():
            pooled = sums[...] / jnp.maximum(cnts[...], 1.0)
            t = jnp.maximum(
                jnp.dot(pooled, w1_ref[...],
                        preferred_element_type=jnp.float32) + b1_ref[...],
                0.0)
            logits = jnp.dot(t, w2_ref[...],
                             preferred_element_type=jnp.float32) + b2_ref[...]
            m = jnp.max(logits, axis=-1, keepdims=True)
            e = jnp.exp(logits - m)
            o_ref[...] = (logits - m) - jnp.log(
                jnp.sum(e, axis=-1, keepdims=True))

    return pl.pallas_call(
        body,
        grid=(nb,),
        in_specs=[
            pl.BlockSpec((NC, BN, H), lambda i: (0, i, 0)),
            pl.BlockSpec((BN, H), lambda i: (i, 0)),
            pl.BlockSpec((BN, H), lambda i: (i, 0)),
            pl.BlockSpec((H, H), lambda i: (0, 0)),
            pl.BlockSpec((H, H), lambda i: (0, 0)),
            pl.BlockSpec((1, H), lambda i: (0, 0)),
            pl.BlockSpec((BN, 1), lambda i: (i, 0)),
            pl.BlockSpec((H, H), lambda i: (0, 0)),
            pl.BlockSpec((1, H), lambda i: (0, 0)),
            pl.BlockSpec((H, C), lambda i: (0, 0)),
            pl.BlockSpec((1, C), lambda i: (0, 0)),
        ],
        out_specs=pl.BlockSpec((G, C), lambda i: (0, 0)),
        out_shape=jax.ShapeDtypeStruct((G, C), jnp.float32),
        scratch_shapes=[
            pltpu.VMEM((G, H), jnp.float32),
            pltpu.VMEM((G, 1), jnp.float32),
        ],
    )


def kernel(x, edge_index, batch, Wl0, bl0, Wr0, Wl1, bl1, Wr1, Wl2, bl2, Wr2,
           W1, b1, W2, b2):
    N, H = x.shape
    C = W2.shape[1]
    E = edge_index.shape[1]
    K = 80
    assert E % (NW * K) == 0
    NCH = E // (NW * K)

    src = edge_index[0].reshape(NW, NCH * K)
    dst = edge_index[1].reshape(NW, NCH, K)

    agg = _make_agg(N, H, K, NCH)
    BE = 6400
    assert E % BE == 0
    NA = -(-((N + 127) // 128) // 8) * 8
    deghist = _make_deghist(E // BE, BE, NA)
    update0 = _make_update0(N, H, BN=400)
    update = _make_update(N, H, BN=400)
    update_pool = _make_update_pool(N, H, C, BN=400)

    (part,) = agg(x, src, dst)
    dh = deghist(edge_index[1].reshape(E // BE, 1, BE))
    degcol = dh.reshape(-1)[:N].reshape(N, 1)
    h, rdeg = update0(part, degcol, x, Wl0, Wr0, bl0.reshape(1, H))
    (part,) = agg(h, src, dst)
    h = update(part, rdeg, h, Wl1, Wr1, bl1.reshape(1, H))
    (part,) = agg(h, src, dst)
    return update_pool(part, rdeg, h, Wl2, Wr2, bl2.reshape(1, H),
                       batch.reshape(N, 1), W1, b1.reshape(1, H), W2,
                       b2.reshape(1, C))


# speedup vs baseline: 1.2362x; 1.2362x over previous
import jax
import jax.numpy as jnp
from jax import lax
from jax.experimental import pallas as pl
from jax.experimental.pallas import tpu as pltpu
from jax.experimental.pallas import tpu_sc as plsc

NC = 2
NS = 16
NW = NC * NS
LANES = 16
G = 64


def _fill_f32(ref, rows, cols, val):
    zv = jnp.full((LANES,), val, jnp.float32)

    def bi(i, carry):
        def bj(j, c):
            ref[i, pl.ds(j * LANES, LANES)] = zv
            return c

        return lax.fori_loop(0, cols // LANES, bj, carry)

    lax.fori_loop(0, rows, bi, 0)


def _strided_chunks(s, nzch, fn):
    def step(i, carry):
        k = s + i * NS

        @pl.when(k < nzch)
        def _():
            fn(k)

        return carry

    lax.fori_loop(0, (nzch + NS - 1) // NS, step, 0)


def _make_agg(N, H, K, NCH, deg_too=False):
    assert N % K == 0 and K == 80
    nzch = N // K
    mesh = plsc.VectorSubcoreMesh(core_axis_name="c", subcore_axis_name="s")
    out_type = [jax.ShapeDtypeStruct((NC, N, H), jnp.float32)]
    if deg_too:
        out_type.append(jax.ShapeDtypeStruct((NC, N, H), jnp.float32))

    def body(*refs):
        if deg_too:
            (h_hbm, src_hbm, dst_hbm, part_hbm, degp_hbm, src_v, dst_v,
             rows0, acc_sh, rows1, sem0, sem1, sems0, sems1) = refs
        else:
            (h_hbm, src_hbm, dst_hbm, part_hbm, src_v, dst_v, rows0,
             acc_sh, rows1, sem0, sem1, sems0, sems1) = refs
        c = lax.axis_index("c")
        s = lax.axis_index("s")
        w = c * NS + s

        a_src = pltpu.async_copy(src_hbm.at[w], src_v, sem0)
        a_dst = pltpu.async_copy(dst_hbm.at[w], dst_v, sem1)

        def zero_acc():
            _strided_chunks(
                s, nzch,
                lambda k: pltpu.sync_copy(rows0, acc_sh.at[pl.ds(k * K, K)]))

        _fill_f32(rows0, K, H, 0.0)
        zero_acc()

        if deg_too:
            _fill_f32(rows1, K, H, 1.0)
            a_src.wait()
            a_dst.wait()
            plsc.subcore_barrier()

            def dchunk(j, carry):
                pltpu.sync_copy(rows1, acc_sh.at[dst_v.at[j]], add=True)
                return carry

            lax.fori_loop(0, NCH, dchunk, 0)
            plsc.subcore_barrier()
            _strided_chunks(
                s, nzch,
                lambda k: pltpu.sync_copy(acc_sh.at[pl.ds(k * K, K)],
                                          degp_hbm.at[c].at[pl.ds(k * K, K)]))
            zero_acc()
        else:
            a_src.wait()
            a_dst.wait()
        plsc.subcore_barrier()

        def gath(j, buf, sem):
            for off, ln in ((0, 24), (24, 24), (48, 16), (64, 16)):
                pltpu.async_copy(h_hbm.at[src_v.at[pl.ds(j * K + off, ln)]],
                                 buf.at[pl.ds(off, ln)], sem)

        def gwait(buf, sem):
            pltpu.make_async_copy(h_hbm.at[pl.ds(0, K)], buf, sem).wait()

        gath(0, rows0, sem0)

        def dbody(t, carry):
            jj = 2 * t
            gath(jj + 1, rows1, sem1)
            gwait(rows0, sem0)
            pltpu.sync_copy(rows0, acc_sh.at[dst_v.at[jj]], add=True)

            @pl.when(jj + 2 < NCH)
            def _():
                gath(jj + 2, rows0, sem0)

            gwait(rows1, sem1)
            pltpu.sync_copy(rows1, acc_sh.at[dst_v.at[jj + 1]], add=True)
            return carry

        lax.fori_loop(0, NCH // 2, dbody, 0)
        if NCH % 2 == 1:
            gwait(rows0, sem0)
            pltpu.sync_copy(rows0, acc_sh.at[dst_v.at[NCH - 1]], add=True)
        plsc.subcore_barrier()

        _strided_chunks(
            s, nzch,
            lambda k: pltpu.sync_copy(acc_sh.at[pl.ds(k * K, K)],
                                      part_hbm.at[c].at[pl.ds(k * K, K)]))

    return pl.kernel(
        body,
        out_type=out_type,
        mesh=mesh,
        scratch_types=[
            pltpu.VMEM((NCH * K,), jnp.int32),
            pltpu.VMEM((NCH, K), jnp.int32),
            pltpu.VMEM((K, H), jnp.float32),
            pltpu.VMEM_SHARED((N, H), jnp.float32),
            pltpu.VMEM((K, H), jnp.float32),
            pltpu.SemaphoreType.DMA,
            pltpu.SemaphoreType.DMA,
            pltpu.SemaphoreType.DMA,
            pltpu.SemaphoreType.DMA,
        ])


def _make_deghist(NBLK, BE, NA):
    def body(d_ref, o_ref, acc):
        i = pl.program_id(0)

        @pl.when(i == 0)
        def _():
            acc[...] = jnp.zeros_like(acc)

        d = d_ref[0, 0, :]
        a = lax.shift_right_logical(d, 7)
        b = jnp.bitwise_and(d, 127)
        oa = (lax.broadcasted_iota(jnp.int32, (NA, BE), 0)
              == a[None, :]).astype(jnp.bfloat16)
        ob = (lax.broadcasted_iota(jnp.int32, (BE, 128), 1)
              == b[:, None]).astype(jnp.bfloat16)
        acc[...] += jnp.dot(oa, ob, preferred_element_type=jnp.float32)

        @pl.when(i == NBLK - 1)
        def _():
            o_ref[...] = acc[...]

    return pl.pallas_call(
        body,
        grid=(NBLK,),
        in_specs=[pl.BlockSpec((1, 1, BE), lambda i: (i, 0, 0))],
        out_specs=pl.BlockSpec((NA, 128), lambda i: (0, 0)),
        out_shape=jax.ShapeDtypeStruct((NA, 128), jnp.float32),
        scratch_shapes=[pltpu.VMEM((NA, 128), jnp.float32)],
    )


def _make_update0(N, H, BN):
    grid = (N // BN,)

    def body(part_ref, deg_ref, h_ref, wl_ref, wr_ref, b_ref, o_ref,
             rdeg_ref):
        rdeg = 1.0 / jnp.maximum(deg_ref[...], 1.0)
        rdeg_ref[...] = jnp.broadcast_to(rdeg, rdeg_ref.shape)
        agg = (part_ref[0] + part_ref[1]) * rdeg
        acc = jnp.dot(agg, wl_ref[...], preferred_element_type=jnp.float32)
        acc = acc + jnp.dot(h_ref[...], wr_ref[...],
                            preferred_element_type=jnp.float32)
        o_ref[...] = jnp.maximum(acc + b_ref[...], 0.0)

    return pl.pallas_call(
        body,
        grid=grid,
        in_specs=[
            pl.BlockSpec((NC, BN, H), lambda i: (0, i, 0)),
            pl.BlockSpec((BN, 1), lambda i: (i, 0)),
            pl.BlockSpec((BN, H), lambda i: (i, 0)),
            pl.BlockSpec((H, H), lambda i: (0, 0)),
            pl.BlockSpec((H, H), lambda i: (0, 0)),
            pl.BlockSpec((1, H), lambda i: (0, 0)),
        ],
        out_specs=[
            pl.BlockSpec((BN, H), lambda i: (i, 0)),
            pl.BlockSpec((BN, H), lambda i: (i, 0)),
        ],
        out_shape=[
            jax.ShapeDtypeStruct((N, H), jnp.float32),
            jax.ShapeDtypeStruct((N, H), jnp.float32),
        ],
    )


def _make_update(N, H, BN):
    grid = (N // BN,)

    def body(part_ref, rdeg_ref, h_ref, wl_ref, wr_ref, b_ref, o_ref):
        agg = (part_ref[0] + part_ref[1]) * rdeg_ref[:, :1]
        acc = jnp.dot(agg, wl_ref[...], preferred_element_type=jnp.float32)
        acc = acc + jnp.dot(h_ref[...], wr_ref[...],
                            preferred_element_type=jnp.float32)
        o_ref[...] = jnp.maximum(acc + b_ref[...], 0.0)

    return pl.pallas_call(
        body,
        grid=grid,
        in_specs=[
            pl.BlockSpec((NC, BN, H), lambda i: (0, i, 0)),
            pl.BlockSpec((BN, H), lambda i: (i, 0)),
            pl.BlockSpec((BN, H), lambda i: (i, 0)),
            pl.BlockSpec((H, H), lambda i: (0, 0)),
            pl.BlockSpec((H, H), lambda i: (0, 0)),
            pl.BlockSpec((1, H), lambda i: (0, 0)),
        ],
        out_specs=pl.BlockSpec((BN, H), lambda i: (i, 0)),
        out_shape=jax.ShapeDtypeStruct((N, H), jnp.float32),
    )


def _make_update_pool(N, H, C, BN):
    nb = N // BN

    def body(part_ref, rdeg_ref, h_ref, wl_ref, wr_ref, b_ref, bt_ref,
             w1_ref, b1_ref, w2_ref, b2_ref, o_ref, sums, cnts):
        i = pl.program_id(0)

        @pl.when(i == 0)
        def _():
            sums[...] = jnp.zeros_like(sums)
            cnts[...] = jnp.zeros_like(cnts)

        agg = (part_ref[0] + part_ref[1]) * rdeg_ref[:, :1]
        acc = jnp.dot(agg, wl_ref[...], preferred_element_type=jnp.float32)
        acc = acc + jnp.dot(h_ref[...], wr_ref[...],
                            preferred_element_type=jnp.float32)
        h3 = jnp.maximum(acc + b_ref[...], 0.0)

        bt = bt_ref[...][:, 0]
        onehot = (lax.broadcasted_iota(jnp.int32, (G, BN), 0)
                  == bt[None, :]).astype(jnp.float32)
        sums[...] += jnp.dot(onehot, h3, preferred_element_type=jnp.float32)
        cnts[...] += jnp.sum(onehot, axis=1, keepdims=True)

        @pl.when(i == nb - 1)
        def _():
            pooled = sums[...] / jnp.maximum(cnts[...], 1.0)
            t = jnp.maximum(
                jnp.dot(pooled, w1_ref[...],
                        preferred_element_type=jnp.float32) + b1_ref[...],
                0.0)
            logits = jnp.dot(t, w2_ref[...],
                             preferred_element_type=jnp.float32) + b2_ref[...]
            m = jnp.max(logits, axis=-1, keepdims=True)
            e = jnp.exp(logits - m)
            o_ref[...] = (logits - m) - jnp.log(
                jnp.sum(e, axis=-1, keepdims=True))

    return pl.pallas_call(
        body,
        grid=(nb,),
        in_specs=[
            pl.BlockSpec((NC, BN, H), lambda i: (0, i, 0)),
            pl.BlockSpec((BN, H), lambda i: (i, 0)),
            pl.BlockSpec((BN, H), lambda i: (i, 0)),
            pl.BlockSpec((H, H), lambda i: (0, 0)),
            pl.BlockSpec((H, H), lambda i: (0, 0)),
            pl.BlockSpec((1, H), lambda i: (0, 0)),
            pl.BlockSpec((BN, 1), lambda i: (i, 0)),
            pl.BlockSpec((H, H), lambda i: (0, 0)),
            pl.BlockSpec((1, H), lambda i: (0, 0)),
            pl.BlockSpec((H, C), lambda i: (0, 0)),
            pl.BlockSpec((1, C), lambda i: (0, 0)),
        ],
        out_specs=pl.BlockSpec((G, C), lambda i: (0, 0)),
        out_shape=jax.ShapeDtypeStruct((G, C), jnp.float32),
        scratch_shapes=[
            pltpu.VMEM((G, H), jnp.float32),
            pltpu.VMEM((G, 1), jnp.float32),
        ],
    )


def kernel(x, edge_index, batch, Wl0, bl0, Wr0, Wl1, bl1, Wr1, Wl2, bl2, Wr2,
           W1, b1, W2, b2):
    N, H = x.shape
    C = W2.shape[1]
    E = edge_index.shape[1]
    K = 80
    assert E % (NW * K) == 0
    NCH = E // (NW * K)

    src = edge_index[0].reshape(NW, NCH * K)
    dst = edge_index[1].reshape(NW, NCH, K)

    agg = _make_agg(N, H, K, NCH)
    BE = 6400
    assert E % BE == 0
    NA = -(-((N + 127) // 128) // 8) * 8
    deghist = _make_deghist(E // BE, BE, NA)
    update0 = _make_update0(N, H, BN=400)
    update = _make_update(N, H, BN=400)
    update_pool = _make_update_pool(N, H, C, BN=400)

    (part,) = agg(x, src, dst)
    dh = deghist(edge_index[1].reshape(E // BE, 1, BE))
    degcol = dh.reshape(-1)[:N].reshape(N, 1)
    h, rdeg = update0(part, degcol, x, Wl0, Wr0, bl0.reshape(1, H))
    (part,) = agg(h, src, dst)
    h = update(part, rdeg, h, Wl1, Wr1, bl1.reshape(1, H))
    (part,) = agg(h, src, dst)
    return update_pool(part, rdeg, h, Wl2, Wr2, bl2.reshape(1, H),
                       batch.reshape(N, 1), W1, b1.reshape(1, H), W2,
                       b2.reshape(1, C))


# speedup vs baseline: 1.3366x; 1.0812x over previous
import jax
import jax.numpy as jnp
from jax import lax
from jax.experimental import pallas as pl
from jax.experimental.pallas import tpu as pltpu
from jax.experimental.pallas import tpu_sc as plsc

NC = 2
NS = 16
NW = NC * NS
LANES = 16
G = 64


def _fill_f32(ref, rows, cols, val):
    zv = jnp.full((LANES,), val, jnp.float32)

    def bi(i, carry):
        def bj(j, c):
            ref[i, pl.ds(j * LANES, LANES)] = zv
            return c

        return lax.fori_loop(0, cols // LANES, bj, carry)

    lax.fori_loop(0, rows, bi, 0)


def _strided_chunks(s, nzch, fn):
    def step(i, carry):
        k = s + i * NS

        @pl.when(k < nzch)
        def _():
            fn(k)

        return carry

    lax.fori_loop(0, (nzch + NS - 1) // NS, step, 0)


def _make_agg(N, H, K, NCH, deg_too=False):
    assert N % K == 0 and K == 80
    nzch = N // K
    mesh = plsc.VectorSubcoreMesh(core_axis_name="c", subcore_axis_name="s")
    out_type = [jax.ShapeDtypeStruct((NC, N, H), jnp.float32)]
    if deg_too:
        out_type.append(jax.ShapeDtypeStruct((NC, N, H), jnp.float32))

    def body(*refs):
        if deg_too:
            (h_hbm, src_hbm, dst_hbm, part_hbm, degp_hbm, src_v, dst_v,
             rows0, acc_sh, rows1, sem0, sem1, sems0, sems1) = refs
        else:
            (h_hbm, src_hbm, dst_hbm, part_hbm, src_v, dst_v, rows0,
             acc_sh, rows1, sem0, sem1, sems0, sems1) = refs
        c = lax.axis_index("c")
        s = lax.axis_index("s")
        w = c * NS + s

        a_src = pltpu.async_copy(src_hbm.at[w], src_v, sem0)
        a_dst = pltpu.async_copy(dst_hbm.at[w], dst_v, sem1)

        def zero_acc():
            _strided_chunks(
                s, nzch,
                lambda k: pltpu.sync_copy(rows0, acc_sh.at[pl.ds(k * K, K)]))

        _fill_f32(rows0, K, H, 0.0)
        zero_acc()

        if deg_too:
            _fill_f32(rows1, K, H, 1.0)
            a_src.wait()
            a_dst.wait()
            plsc.subcore_barrier()

            def dchunk(j, carry):
                pltpu.sync_copy(rows1, acc_sh.at[dst_v.at[j]], add=True)
                return carry

            lax.fori_loop(0, NCH, dchunk, 0)
            plsc.subcore_barrier()
            _strided_chunks(
                s, nzch,
                lambda k: pltpu.sync_copy(acc_sh.at[pl.ds(k * K, K)],
                                          degp_hbm.at[c].at[pl.ds(k * K, K)]))
            zero_acc()
        else:
            a_src.wait()
            a_dst.wait()
        plsc.subcore_barrier()

        def gath(j, buf, sem):
            for off, ln in ((0, 24), (24, 24), (48, 16), (64, 16)):
                pltpu.async_copy(h_hbm.at[src_v.at[pl.ds(j * K + off, ln)]],
                                 buf.at[pl.ds(off, ln)], sem)

        def gwait(buf, sem):
            pltpu.make_async_copy(h_hbm.at[pl.ds(0, K)], buf, sem).wait()

        gath(0, rows0, sem0)

        def dbody(t, carry):
            jj = 2 * t
            gath(jj + 1, rows1, sem1)
            gwait(rows0, sem0)
            pltpu.sync_copy(rows0, acc_sh.at[dst_v.at[jj]], add=True)

            @pl.when(jj + 2 < NCH)
            def _():
                gath(jj + 2, rows0, sem0)

            gwait(rows1, sem1)
            pltpu.sync_copy(rows1, acc_sh.at[dst_v.at[jj + 1]], add=True)
            return carry

        lax.fori_loop(0, NCH // 2, dbody, 0)
        if NCH % 2 == 1:
            gwait(rows0, sem0)
            pltpu.sync_copy(rows0, acc_sh.at[dst_v.at[NCH - 1]], add=True)
        plsc.subcore_barrier()

        _strided_chunks(
            s, nzch,
            lambda k: pltpu.sync_copy(acc_sh.at[pl.ds(k * K, K)],
                                      part_hbm.at[c].at[pl.ds(k * K, K)]))

    return pl.kernel(
        body,
        out_type=out_type,
        mesh=mesh,
        scratch_types=[
            pltpu.VMEM((NCH * K,), jnp.int32),
            pltpu.VMEM((NCH, K), jnp.int32),
            pltpu.VMEM((K, H), jnp.float32),
            pltpu.VMEM_SHARED((N, H), jnp.float32),
            pltpu.VMEM((K, H), jnp.float32),
            pltpu.SemaphoreType.DMA,
            pltpu.SemaphoreType.DMA,
            pltpu.SemaphoreType.DMA,
            pltpu.SemaphoreType.DMA,
        ])


def _make_deghist(NBLK, BE, NA):
    def body(d_ref, o_ref, acc):
        i = pl.program_id(0)

        @pl.when(i == 0)
        def _():
            acc[...] = jnp.zeros_like(acc)

        d = d_ref[0, 0, :]
        a = lax.shift_right_logical(d, 7)
        b = jnp.bitwise_and(d, 127)
        oa = (lax.broadcasted_iota(jnp.int32, (NA, BE), 0)
              == a[None, :]).astype(jnp.bfloat16)
        ob = (lax.broadcasted_iota(jnp.int32, (BE, 128), 1)
              == b[:, None]).astype(jnp.bfloat16)
        acc[...] += jnp.dot(oa, ob, preferred_element_type=jnp.float32)

        @pl.when(i == NBLK - 1)
        def _():
            o_ref[...] = acc[...]

    return pl.pallas_call(
        body,
        grid=(NBLK,),
        in_specs=[pl.BlockSpec((1, 1, BE), lambda i: (i, 0, 0))],
        out_specs=pl.BlockSpec((NA, 128), lambda i: (0, 0)),
        out_shape=jax.ShapeDtypeStruct((NA, 128), jnp.float32),
        scratch_shapes=[pltpu.VMEM((NA, 128), jnp.float32)],
    )


def _make_update0(N, H, BN):
    grid = (N // BN,)

    def body(part_ref, deg_ref, h_ref, wl_ref, wr_ref, b_ref, o_ref,
             rdeg_ref):
        rdeg = 1.0 / jnp.maximum(deg_ref[...], 1.0)
        rdeg_ref[...] = jnp.broadcast_to(rdeg, rdeg_ref.shape)
        agg = (part_ref[0] + part_ref[1]) * rdeg
        acc = jnp.dot(agg, wl_ref[...], preferred_element_type=jnp.float32)
        acc = acc + jnp.dot(h_ref[...], wr_ref[...],
                            preferred_element_type=jnp.float32)
        o_ref[...] = jnp.maximum(acc + b_ref[...], 0.0)

    return pl.pallas_call(
        body,
        grid=grid,
        in_specs=[
            pl.BlockSpec((NC, BN, H), lambda i: (0, i, 0)),
            pl.BlockSpec((BN, 1), lambda i: (i, 0)),
            pl.BlockSpec((BN, H), lambda i: (i, 0)),
            pl.BlockSpec((H, H), lambda i: (0, 0)),
            pl.BlockSpec((H, H), lambda i: (0, 0)),
            pl.BlockSpec((1, H), lambda i: (0, 0)),
        ],
        out_specs=[
            pl.BlockSpec((BN, H), lambda i: (i, 0)),
            pl.BlockSpec((BN, H), lambda i: (i, 0)),
        ],
        out_shape=[
            jax.ShapeDtypeStruct((N, H), jnp.float32),
            jax.ShapeDtypeStruct((N, H), jnp.float32),
        ],
    )


def _make_update(N, H, BN):
    grid = (N // BN,)

    def body(part_ref, rdeg_ref, h_ref, wl_ref, wr_ref, b_ref, o_ref):
        agg = (part_ref[0] + part_ref[1]) * rdeg_ref[:, :1]
        acc = jnp.dot(agg, wl_ref[...], preferred_element_type=jnp.float32)
        acc = acc + jnp.dot(h_ref[...], wr_ref[...],
                            preferred_element_type=jnp.float32)
        o_ref[...] = jnp.maximum(acc + b_ref[...], 0.0)

    return pl.pallas_call(
        body,
        grid=grid,
        in_specs=[
            pl.BlockSpec((NC, BN, H), lambda i: (0, i, 0)),
            pl.BlockSpec((BN, H), lambda i: (i, 0)),
            pl.BlockSpec((BN, H), lambda i: (i, 0)),
            pl.BlockSpec((H, H), lambda i: (0, 0)),
            pl.BlockSpec((H, H), lambda i: (0, 0)),
            pl.BlockSpec((1, H), lambda i: (0, 0)),
        ],
        out_specs=pl.BlockSpec((BN, H), lambda i: (i, 0)),
        out_shape=jax.ShapeDtypeStruct((N, H), jnp.float32),
    )


def _make_update_pool(N, H, C, BN):
    nb = N // BN

    def body(part_ref, rdeg_ref, h_ref, wl_ref, wr_ref, b_ref, bt_ref,
             w1_ref, b1_ref, w2_ref, b2_ref, o_ref, sums, cnts):
        i = pl.program_id(0)

        @pl.when(i == 0)
        def _():
            sums[...] = jnp.zeros_like(sums)
            cnts[...] = jnp.zeros_like(cnts)

        agg = (part_ref[0] + part_ref[1]) * rdeg_ref[:, :1]
        acc = jnp.dot(agg, wl_ref[...], preferred_element_type=jnp.float32)
        acc = acc + jnp.dot(h_ref[...], wr_ref[...],
                            preferred_element_type=jnp.float32)
        h3 = jnp.maximum(acc + b_ref[...], 0.0)

        bt = bt_ref[...][:, 0]
        onehot = (lax.broadcasted_iota(jnp.int32, (G, BN), 0)
                  == bt[None, :]).astype(jnp.float32)
        sums[...] += jnp.dot(onehot, h3, preferred_element_type=jnp.float32)
        cnts[...] += jnp.sum(onehot, axis=1, keepdims=True)

        @pl.when(i == nb - 1)
        def _():
            pooled = sums[...] / jnp.maximum(cnts[...], 1.0)
            t = jnp.maximum(
                jnp.dot(pooled, w1_ref[...],
                        preferred_element_type=jnp.float32) + b1_ref[...],
                0.0)
            logits = jnp.dot(t, w2_ref[...],
                             preferred_element_type=jnp.float32) + b2_ref[...]
            m = jnp.max(logits, axis=-1, keepdims=True)
            e = jnp.exp(logits - m)
            o_ref[...] = (logits - m) - jnp.log(
                jnp.sum(e, axis=-1, keepdims=True))

    return pl.pallas_call(
        body,
        grid=(nb,),
        in_specs=[
            pl.BlockSpec((NC, BN, H), lambda i: (0, i, 0)),
            pl.BlockSpec((BN, H), lambda i: (i, 0)),
            pl.BlockSpec((BN, H), lambda i: (i, 0)),
            pl.BlockSpec((H, H), lambda i: (0, 0)),
            pl.BlockSpec((H, H), lambda i: (0, 0)),
            pl.BlockSpec((1, H), lambda i: (0, 0)),
            pl.BlockSpec((BN, 1), lambda i: (i, 0)),
            pl.BlockSpec((H, H), lambda i: (0, 0)),
            pl.BlockSpec((1, H), lambda i: (0, 0)),
            pl.BlockSpec((H, C), lambda i: (0, 0)),
            pl.BlockSpec((1, C), lambda i: (0, 0)),
        ],
        out_specs=pl.BlockSpec((G, C), lambda i: (0, 0)),
        out_shape=jax.ShapeDtypeStruct((G, C), jnp.float32),
        scratch_shapes=[
            pltpu.VMEM((G, H), jnp.float32),
            pltpu.VMEM((G, 1), jnp.float32),
        ],
    )


def kernel(x, edge_index, batch, Wl0, bl0, Wr0, Wl1, bl1, Wr1, Wl2, bl2, Wr2,
           W1, b1, W2, b2):
    N, H = x.shape
    C = W2.shape[1]
    E = edge_index.shape[1]
    K = 80
    assert E % (NW * K) == 0
    NCH = E // (NW * K)

    src = edge_index[0].reshape(NW, NCH * K)
    dst = edge_index[1].reshape(NW, NCH, K)

    agg = _make_agg(N, H, K, NCH)
    BE = 12800
    assert E % BE == 0
    NA = -(-((N + 127) // 128) // 8) * 8
    deghist = _make_deghist(E // BE, BE, NA)
    update0 = _make_update0(N, H, BN=2000)
    update = _make_update(N, H, BN=2000)
    update_pool = _make_update_pool(N, H, C, BN=2000)

    (part,) = agg(x, src, dst)
    dh = deghist(edge_index[1].reshape(E // BE, 1, BE))
    degcol = dh.reshape(-1)[:N].reshape(N, 1)
    h, rdeg = update0(part, degcol, x, Wl0, Wr0, bl0.reshape(1, H))
    (part,) = agg(h, src, dst)
    h = update(part, rdeg, h, Wl1, Wr1, bl1.reshape(1, H))
    (part,) = agg(h, src, dst)
    return update_pool(part, rdeg, h, Wl2, Wr2, bl2.reshape(1, H),
                       batch.reshape(N, 1), W1, b1.reshape(1, H), W2,
                       b2.reshape(1, C))


# speedup vs baseline: 1.3423x; 1.0043x over previous
import jax
import jax.numpy as jnp
from jax import lax
from jax.experimental import pallas as pl
from jax.experimental.pallas import tpu as pltpu
from jax.experimental.pallas import tpu_sc as plsc

NC = 2
NS = 16
NW = NC * NS
LANES = 16
G = 64


def _fill_f32(ref, rows, cols, val):
    zv = jnp.full((LANES,), val, jnp.float32)

    def bi(i, carry):
        def bj(j, c):
            ref[i, pl.ds(j * LANES, LANES)] = zv
            return c

        return lax.fori_loop(0, cols // LANES, bj, carry)

    lax.fori_loop(0, rows, bi, 0)


def _strided_chunks(s, nzch, fn):
    def step(i, carry):
        k = s + i * NS

        @pl.when(k < nzch)
        def _():
            fn(k)

        return carry

    lax.fori_loop(0, (nzch + NS - 1) // NS, step, 0)


def _make_agg(N, H, K, NCH, deg_too=False):
    assert N % K == 0 and K == 80
    nzch = N // K
    mesh = plsc.VectorSubcoreMesh(core_axis_name="c", subcore_axis_name="s")
    out_type = [jax.ShapeDtypeStruct((NC, N, H), jnp.float32)]
    if deg_too:
        out_type.append(jax.ShapeDtypeStruct((NC, N, H), jnp.float32))

    def body(*refs):
        if deg_too:
            (h_hbm, src_hbm, dst_hbm, part_hbm, degp_hbm, src_v, dst_v,
             rows0, acc_sh, rows1, sem0, sem1, sems0, sems1) = refs
        else:
            (h_hbm, src_hbm, dst_hbm, part_hbm, src_v, dst_v, rows0,
             acc_sh, rows1, sem0, sem1, sems0, sems1) = refs
        c = lax.axis_index("c")
        s = lax.axis_index("s")
        w = c * NS + s

        a_src = pltpu.async_copy(src_hbm.at[w], src_v, sem0)
        a_dst = pltpu.async_copy(dst_hbm.at[w], dst_v, sem1)

        def zero_acc():
            _strided_chunks(
                s, nzch,
                lambda k: pltpu.sync_copy(rows0, acc_sh.at[pl.ds(k * K, K)]))

        _fill_f32(rows0, K, H, 0.0)
        zero_acc()

        if deg_too:
            _fill_f32(rows1, K, H, 1.0)
            a_src.wait()
            a_dst.wait()
            plsc.subcore_barrier()

            def dchunk(j, carry):
                pltpu.sync_copy(rows1, acc_sh.at[dst_v.at[j]], add=True)
                return carry

            lax.fori_loop(0, NCH, dchunk, 0)
            plsc.subcore_barrier()
            _strided_chunks(
                s, nzch,
                lambda k: pltpu.sync_copy(acc_sh.at[pl.ds(k * K, K)],
                                          degp_hbm.at[c].at[pl.ds(k * K, K)]))
            zero_acc()
        else:
            a_src.wait()
            a_dst.wait()
        plsc.subcore_barrier()

        def gath(j, buf, sem):
            for off, ln in ((0, 24), (24, 24), (48, 16), (64, 16)):
                pltpu.async_copy(h_hbm.at[src_v.at[pl.ds(j * K + off, ln)]],
                                 buf.at[pl.ds(off, ln)], sem)

        def gwait(buf, sem):
            pltpu.make_async_copy(h_hbm.at[pl.ds(0, K)], buf, sem).wait()

        gath(0, rows0, sem0)

        def dbody(t, carry):
            jj = 2 * t
            gath(jj + 1, rows1, sem1)
            gwait(rows0, sem0)
            pltpu.sync_copy(rows0, acc_sh.at[dst_v.at[jj]], add=True)

            @pl.when(jj + 2 < NCH)
            def _():
                gath(jj + 2, rows0, sem0)

            gwait(rows1, sem1)
            pltpu.sync_copy(rows1, acc_sh.at[dst_v.at[jj + 1]], add=True)
            return carry

        lax.fori_loop(0, NCH // 2, dbody, 0)
        if NCH % 2 == 1:
            gwait(rows0, sem0)
            pltpu.sync_copy(rows0, acc_sh.at[dst_v.at[NCH - 1]], add=True)
        plsc.subcore_barrier()

        _strided_chunks(
            s, nzch,
            lambda k: pltpu.sync_copy(acc_sh.at[pl.ds(k * K, K)],
                                      part_hbm.at[c].at[pl.ds(k * K, K)]))

    return pl.kernel(
        body,
        out_type=out_type,
        mesh=mesh,
        scratch_types=[
            pltpu.VMEM((NCH * K,), jnp.int32),
            pltpu.VMEM((NCH, K), jnp.int32),
            pltpu.VMEM((K, H), jnp.float32),
            pltpu.VMEM_SHARED((N, H), jnp.float32),
            pltpu.VMEM((K, H), jnp.float32),
            pltpu.SemaphoreType.DMA,
            pltpu.SemaphoreType.DMA,
            pltpu.SemaphoreType.DMA,
            pltpu.SemaphoreType.DMA,
        ])


def _make_deghist(NBLK, BE, NA):
    def body(d_ref, o_ref, acc):
        i = pl.program_id(0)

        @pl.when(i == 0)
        def _():
            acc[...] = jnp.zeros_like(acc)

        d = d_ref[0, 0, :]
        a = lax.shift_right_logical(d, 7)
        b = jnp.bitwise_and(d, 127)
        oa = (lax.broadcasted_iota(jnp.int32, (NA, BE), 0)
              == a[None, :]).astype(jnp.bfloat16)
        ob = (lax.broadcasted_iota(jnp.int32, (BE, 128), 1)
              == b[:, None]).astype(jnp.bfloat16)
        acc[...] += jnp.dot(oa, ob, preferred_element_type=jnp.float32)

        @pl.when(i == NBLK - 1)
        def _():
            o_ref[...] = acc[...]

    return pl.pallas_call(
        body,
        grid=(NBLK,),
        in_specs=[pl.BlockSpec((1, 1, BE), lambda i: (i, 0, 0))],
        out_specs=pl.BlockSpec((NA, 128), lambda i: (0, 0)),
        out_shape=jax.ShapeDtypeStruct((NA, 128), jnp.float32),
        scratch_shapes=[pltpu.VMEM((NA, 128), jnp.float32)],
    )


def _make_update0(N, H, BN):
    grid = (N // BN,)

    def body(part_ref, deg_ref, h_ref, wl_ref, wr_ref, b_ref, o_ref,
             rdeg_ref):
        rdeg = 1.0 / jnp.maximum(deg_ref[...], 1.0)
        rdeg_ref[...] = jnp.broadcast_to(rdeg, rdeg_ref.shape)
        agg = (part_ref[0] + part_ref[1]) * rdeg
        acc = jnp.dot(agg, wl_ref[...], preferred_element_type=jnp.float32)
        acc = acc + jnp.dot(h_ref[...], wr_ref[...],
                            preferred_element_type=jnp.float32)
        o_ref[...] = jnp.maximum(acc + b_ref[...], 0.0)

    return pl.pallas_call(
        body,
        grid=grid,
        in_specs=[
            pl.BlockSpec((NC, BN, H), lambda i: (0, i, 0)),
            pl.BlockSpec((BN, 1), lambda i: (i, 0)),
            pl.BlockSpec((BN, H), lambda i: (i, 0)),
            pl.BlockSpec((H, H), lambda i: (0, 0)),
            pl.BlockSpec((H, H), lambda i: (0, 0)),
            pl.BlockSpec((1, H), lambda i: (0, 0)),
        ],
        out_specs=[
            pl.BlockSpec((BN, H), lambda i: (i, 0)),
            pl.BlockSpec((BN, H), lambda i: (i, 0)),
        ],
        out_shape=[
            jax.ShapeDtypeStruct((N, H), jnp.float32),
            jax.ShapeDtypeStruct((N, H), jnp.float32),
        ],
    )


def _make_update(N, H, BN):
    grid = (N // BN,)

    def body(part_ref, rdeg_ref, h_ref, wl_ref, wr_ref, b_ref, o_ref):
        agg = (part_ref[0] + part_ref[1]) * rdeg_ref[:, :1]
        acc = jnp.dot(agg, wl_ref[...], preferred_element_type=jnp.float32)
        acc = acc + jnp.dot(h_ref[...], wr_ref[...],
                            preferred_element_type=jnp.float32)
        o_ref[...] = jnp.maximum(acc + b_ref[...], 0.0)

    return pl.pallas_call(
        body,
        grid=grid,
        in_specs=[
            pl.BlockSpec((NC, BN, H), lambda i: (0, i, 0)),
            pl.BlockSpec((BN, H), lambda i: (i, 0)),
            pl.BlockSpec((BN, H), lambda i: (i, 0)),
            pl.BlockSpec((H, H), lambda i: (0, 0)),
            pl.BlockSpec((H, H), lambda i: (0, 0)),
            pl.BlockSpec((1, H), lambda i: (0, 0)),
        ],
        out_specs=pl.BlockSpec((BN, H), lambda i: (i, 0)),
        out_shape=jax.ShapeDtypeStruct((N, H), jnp.float32),
    )


def _make_update_pool(N, H, C, BN):
    nb = N // BN

    def body(part_ref, rdeg_ref, h_ref, wl_ref, wr_ref, b_ref, bt_ref,
             w1_ref, b1_ref, w2_ref, b2_ref, o_ref, sums, cnts):
        i = pl.program_id(0)

        @pl.when(i == 0)
        def _():
            sums[...] = jnp.zeros_like(sums)
            cnts[...] = jnp.zeros_like(cnts)

        agg = (part_ref[0] + part_ref[1]) * rdeg_ref[:, :1]
        acc = jnp.dot(agg, wl_ref[...], preferred_element_type=jnp.float32)
        acc = acc + jnp.dot(h_ref[...], wr_ref[...],
                            preferred_element_type=jnp.float32)
        h3 = jnp.maximum(acc + b_ref[...], 0.0)

        bt = bt_ref[...][:, 0]
        onehot = (lax.broadcasted_iota(jnp.int32, (G, BN), 0)
                  == bt[None, :]).astype(jnp.float32)
        sums[...] += jnp.dot(onehot, h3, preferred_element_type=jnp.float32)
        cnts[...] += jnp.sum(onehot, axis=1, keepdims=True)

        @pl.when(i == nb - 1)
        def _():
            pooled = sums[...] / jnp.maximum(cnts[...], 1.0)
            t = jnp.maximum(
                jnp.dot(pooled, w1_ref[...],
                        preferred_element_type=jnp.float32) + b1_ref[...],
                0.0)
            logits = jnp.dot(t, w2_ref[...],
                             preferred_element_type=jnp.float32) + b2_ref[...]
            m = jnp.max(logits, axis=-1, keepdims=True)
            e = jnp.exp(logits - m)
            o_ref[...] = (logits - m) - jnp.log(
                jnp.sum(e, axis=-1, keepdims=True))

    return pl.pallas_call(
        body,
        grid=(nb,),
        in_specs=[
            pl.BlockSpec((NC, BN, H), lambda i: (0, i, 0)),
            pl.BlockSpec((BN, H), lambda i: (i, 0)),
            pl.BlockSpec((BN, H), lambda i: (i, 0)),
            pl.BlockSpec((H, H), lambda i: (0, 0)),
            pl.BlockSpec((H, H), lambda i: (0, 0)),
            pl.BlockSpec((1, H), lambda i: (0, 0)),
            pl.BlockSpec((BN, 1), lambda i: (i, 0)),
            pl.BlockSpec((H, H), lambda i: (0, 0)),
            pl.BlockSpec((1, H), lambda i: (0, 0)),
            pl.BlockSpec((H, C), lambda i: (0, 0)),
            pl.BlockSpec((1, C), lambda i: (0, 0)),
        ],
        out_specs=pl.BlockSpec((G, C), lambda i: (0, 0)),
        out_shape=jax.ShapeDtypeStruct((G, C), jnp.float32),
        scratch_shapes=[
            pltpu.VMEM((G, H), jnp.float32),
            pltpu.VMEM((G, 1), jnp.float32),
        ],
    )


def kernel(x, edge_index, batch, Wl0, bl0, Wr0, Wl1, bl1, Wr1, Wl2, bl2, Wr2,
           W1, b1, W2, b2):
    N, H = x.shape
    C = W2.shape[1]
    E = edge_index.shape[1]
    K = 80
    assert E % (NW * K) == 0
    NCH = E // (NW * K)

    src = edge_index[0].reshape(NW, NCH * K)
    dst = edge_index[1].reshape(NW, NCH, K)

    agg = _make_agg(N, H, K, NCH)
    BE = 12800
    assert E % BE == 0
    NA = -(-((N + 127) // 128) // 8) * 8
    deghist = _make_deghist(E // BE, BE, NA)
    update0 = _make_update0(N, H, BN=5000)
    update = _make_update(N, H, BN=5000)
    update_pool = _make_update_pool(N, H, C, BN=2000)

    (part,) = agg(x, src, dst)
    dh = deghist(edge_index[1].reshape(E // BE, 1, BE))
    degcol = dh.reshape(-1)[:N].reshape(N, 1)
    h, rdeg = update0(part, degcol, x, Wl0, Wr0, bl0.reshape(1, H))
    (part,) = agg(h, src, dst)
    h = update(part, rdeg, h, Wl1, Wr1, bl1.reshape(1, H))
    (part,) = agg(h, src, dst)
    return update_pool(part, rdeg, h, Wl2, Wr2, bl2.reshape(1, H),
                       batch.reshape(N, 1), W1, b1.reshape(1, H), W2,
                       b2.reshape(1, C))


# speedup vs baseline: 1.3432x; 1.0007x over previous
import jax
import jax.numpy as jnp
from jax import lax
from jax.experimental import pallas as pl
from jax.experimental.pallas import tpu as pltpu
from jax.experimental.pallas import tpu_sc as plsc

NC = 2
NS = 16
NW = NC * NS
LANES = 16
G = 64


def _fill_f32(ref, rows, cols, val):
    zv = jnp.full((LANES,), val, jnp.float32)

    def bi(i, carry):
        def bj(j, c):
            ref[i, pl.ds(j * LANES, LANES)] = zv
            return c

        return lax.fori_loop(0, cols // LANES, bj, carry)

    lax.fori_loop(0, rows, bi, 0)


def _strided_chunks(s, nzch, fn):
    def step(i, carry):
        k = s + i * NS

        @pl.when(k < nzch)
        def _():
            fn(k)

        return carry

    lax.fori_loop(0, (nzch + NS - 1) // NS, step, 0)


def _make_agg(N, H, K, NCH, deg_too=False):
    assert N % K == 0 and K == 80
    nzch = N // K
    mesh = plsc.VectorSubcoreMesh(core_axis_name="c", subcore_axis_name="s")
    out_type = [jax.ShapeDtypeStruct((NC, N, H), jnp.float32)]
    if deg_too:
        out_type.append(jax.ShapeDtypeStruct((NC, N, H), jnp.float32))

    def body(*refs):
        if deg_too:
            (h_hbm, src_hbm, dst_hbm, part_hbm, degp_hbm, src_v, dst_v,
             rows0, acc_sh, rows1, sem0, sem1, sems0, sems1) = refs
        else:
            (h_hbm, src_hbm, dst_hbm, part_hbm, src_v, dst_v, rows0,
             acc_sh, rows1, sem0, sem1, sems0, sems1) = refs
        c = lax.axis_index("c")
        s = lax.axis_index("s")
        w = c * NS + s

        a_src = pltpu.async_copy(src_hbm.at[w], src_v, sem0)
        a_dst = pltpu.async_copy(dst_hbm.at[w], dst_v, sem1)

        def zero_acc():
            _strided_chunks(
                s, nzch,
                lambda k: pltpu.sync_copy(rows0, acc_sh.at[pl.ds(k * K, K)]))

        _fill_f32(rows0, K, H, 0.0)
        zero_acc()

        if deg_too:
            _fill_f32(rows1, K, H, 1.0)
            a_src.wait()
            a_dst.wait()
            plsc.subcore_barrier()

            def dchunk(j, carry):
                pltpu.sync_copy(rows1, acc_sh.at[dst_v.at[j]], add=True)
                return carry

            lax.fori_loop(0, NCH, dchunk, 0)
            plsc.subcore_barrier()
            _strided_chunks(
                s, nzch,
                lambda k: pltpu.sync_copy(acc_sh.at[pl.ds(k * K, K)],
                                          degp_hbm.at[c].at[pl.ds(k * K, K)]))
            zero_acc()
        else:
            a_src.wait()
            a_dst.wait()
        plsc.subcore_barrier()

        def gath(j, buf, sem):
            for off, ln in ((0, 24), (24, 24), (48, 16), (64, 16)):
                pltpu.async_copy(h_hbm.at[src_v.at[pl.ds(j * K + off, ln)]],
                                 buf.at[pl.ds(off, ln)], sem)

        def gwait(buf, sem):
            pltpu.make_async_copy(h_hbm.at[pl.ds(0, K)], buf, sem).wait()

        gath(0, rows0, sem0)

        def dbody(t, carry):
            jj = 2 * t
            gath(jj + 1, rows1, sem1)
            gwait(rows0, sem0)
            pltpu.sync_copy(rows0, acc_sh.at[dst_v.at[jj]], add=True)

            @pl.when(jj + 2 < NCH)
            def _():
                gath(jj + 2, rows0, sem0)

            gwait(rows1, sem1)
            pltpu.sync_copy(rows1, acc_sh.at[dst_v.at[jj + 1]], add=True)
            return carry

        lax.fori_loop(0, NCH // 2, dbody, 0)
        if NCH % 2 == 1:
            gwait(rows0, sem0)
            pltpu.sync_copy(rows0, acc_sh.at[dst_v.at[NCH - 1]], add=True)
        plsc.subcore_barrier()

        _strided_chunks(
            s, nzch,
            lambda k: pltpu.sync_copy(acc_sh.at[pl.ds(k * K, K)],
                                      part_hbm.at[c].at[pl.ds(k * K, K)]))

    return pl.kernel(
        body,
        out_type=out_type,
        mesh=mesh,
        scratch_types=[
            pltpu.VMEM((NCH * K,), jnp.int32),
            pltpu.VMEM((NCH, K), jnp.int32),
            pltpu.VMEM((K, H), jnp.float32),
            pltpu.VMEM_SHARED((N, H), jnp.float32),
            pltpu.VMEM((K, H), jnp.float32),
            pltpu.SemaphoreType.DMA,
            pltpu.SemaphoreType.DMA,
            pltpu.SemaphoreType.DMA,
            pltpu.SemaphoreType.DMA,
        ])


def _make_deghist(NBLK, BE, NA):
    def body(d_ref, o_ref, acc):
        i = pl.program_id(0)

        @pl.when(i == 0)
        def _():
            acc[...] = jnp.zeros_like(acc)

        d = d_ref[0, 0, :]
        a = lax.shift_right_logical(d, 7)
        b = jnp.bitwise_and(d, 127)
        oa = (lax.broadcasted_iota(jnp.int32, (NA, BE), 0)
              == a[None, :]).astype(jnp.bfloat16)
        ob = (lax.broadcasted_iota(jnp.int32, (BE, 128), 1)
              == b[:, None]).astype(jnp.bfloat16)
        acc[...] += jnp.dot(oa, ob, preferred_element_type=jnp.float32)

        @pl.when(i == NBLK - 1)
        def _():
            o_ref[...] = acc[...]

    return pl.pallas_call(
        body,
        grid=(NBLK,),
        in_specs=[pl.BlockSpec((1, 1, BE), lambda i: (i, 0, 0))],
        out_specs=pl.BlockSpec((NA, 128), lambda i: (0, 0)),
        out_shape=jax.ShapeDtypeStruct((NA, 128), jnp.float32),
        scratch_shapes=[pltpu.VMEM((NA, 128), jnp.float32)],
    )


def _make_update0(N, H, BN):
    grid = (N // BN,)

    def body(part_ref, deg_ref, h_ref, wl_ref, wr_ref, b_ref, o_ref,
             rdeg_ref):
        rdeg = 1.0 / jnp.maximum(deg_ref[...], 1.0)
        rdeg_ref[...] = jnp.broadcast_to(rdeg, rdeg_ref.shape)
        agg = (part_ref[0] + part_ref[1]) * rdeg
        acc = jnp.dot(agg, wl_ref[...], preferred_element_type=jnp.float32)
        acc = acc + jnp.dot(h_ref[...], wr_ref[...],
                            preferred_element_type=jnp.float32)
        o_ref[...] = jnp.maximum(acc + b_ref[...], 0.0)

    return pl.pallas_call(
        body,
        grid=grid,
        in_specs=[
            pl.BlockSpec((NC, BN, H), lambda i: (0, i, 0)),
            pl.BlockSpec((BN, 1), lambda i: (i, 0)),
            pl.BlockSpec((BN, H), lambda i: (i, 0)),
            pl.BlockSpec((H, H), lambda i: (0, 0)),
            pl.BlockSpec((H, H), lambda i: (0, 0)),
            pl.BlockSpec((1, H), lambda i: (0, 0)),
        ],
        out_specs=[
            pl.BlockSpec((BN, H), lambda i: (i, 0)),
            pl.BlockSpec((BN, H), lambda i: (i, 0)),
        ],
        out_shape=[
            jax.ShapeDtypeStruct((N, H), jnp.float32),
            jax.ShapeDtypeStruct((N, H), jnp.float32),
        ],
    )


def _make_update(N, H, BN):
    grid = (N // BN,)

    def body(part_ref, rdeg_ref, h_ref, wl_ref, wr_ref, b_ref, o_ref):
        agg = (part_ref[0] + part_ref[1]) * rdeg_ref[:, :1]
        acc = jnp.dot(agg, wl_ref[...], preferred_element_type=jnp.float32)
        acc = acc + jnp.dot(h_ref[...], wr_ref[...],
                            preferred_element_type=jnp.float32)
        o_ref[...] = jnp.maximum(acc + b_ref[...], 0.0)

    return pl.pallas_call(
        body,
        grid=grid,
        in_specs=[
            pl.BlockSpec((NC, BN, H), lambda i: (0, i, 0)),
            pl.BlockSpec((BN, H), lambda i: (i, 0)),
            pl.BlockSpec((BN, H), lambda i: (i, 0)),
            pl.BlockSpec((H, H), lambda i: (0, 0)),
            pl.BlockSpec((H, H), lambda i: (0, 0)),
            pl.BlockSpec((1, H), lambda i: (0, 0)),
        ],
        out_specs=pl.BlockSpec((BN, H), lambda i: (i, 0)),
        out_shape=jax.ShapeDtypeStruct((N, H), jnp.float32),
    )


def _make_update_pool(N, H, C, BN):
    nb = N // BN

    def body(part_ref, rdeg_ref, h_ref, wl_ref, wr_ref, b_ref, bt_ref,
             w1_ref, b1_ref, w2_ref, b2_ref, o_ref, sums, cnts):
        i = pl.program_id(0)

        @pl.when(i == 0)
        def _():
            sums[...] = jnp.zeros_like(sums)
            cnts[...] = jnp.zeros_like(cnts)

        agg = (part_ref[0] + part_ref[1]) * rdeg_ref[:, :1]
        acc = jnp.dot(agg, wl_ref[...], preferred_element_type=jnp.float32)
        acc = acc + jnp.dot(h_ref[...], wr_ref[...],
                            preferred_element_type=jnp.float32)
        h3 = jnp.maximum(acc + b_ref[...], 0.0)

        bt = bt_ref[...][:, 0]
        onehot = (lax.broadcasted_iota(jnp.int32, (G, BN), 0)
                  == bt[None, :]).astype(jnp.float32)
        sums[...] += jnp.dot(onehot, h3, preferred_element_type=jnp.float32)
        cnts[...] += jnp.sum(onehot, axis=1, keepdims=True)

        @pl.when(i == nb - 1)
        def _():
            pooled = sums[...] / jnp.maximum(cnts[...], 1.0)
            t = jnp.maximum(
                jnp.dot(pooled, w1_ref[...],
                        preferred_element_type=jnp.float32) + b1_ref[...],
                0.0)
            logits = jnp.dot(t, w2_ref[...],
                             preferred_element_type=jnp.float32) + b2_ref[...]
            m = jnp.max(logits, axis=-1, keepdims=True)
            e = jnp.exp(logits - m)
            o_ref[...] = (logits - m) - jnp.log(
                jnp.sum(e, axis=-1, keepdims=True))

    return pl.pallas_call(
        body,
        grid=(nb,),
        in_specs=[
            pl.BlockSpec((NC, BN, H), lambda i: (0, i, 0)),
            pl.BlockSpec((BN, H), lambda i: (i, 0)),
            pl.BlockSpec((BN, H), lambda i: (i, 0)),
            pl.BlockSpec((H, H), lambda i: (0, 0)),
            pl.BlockSpec((H, H), lambda i: (0, 0)),
            pl.BlockSpec((1, H), lambda i: (0, 0)),
            pl.BlockSpec((BN, 1), lambda i: (i, 0)),
            pl.BlockSpec((H, H), lambda i: (0, 0)),
            pl.BlockSpec((1, H), lambda i: (0, 0)),
            pl.BlockSpec((H, C), lambda i: (0, 0)),
            pl.BlockSpec((1, C), lambda i: (0, 0)),
        ],
        out_specs=pl.BlockSpec((G, C), lambda i: (0, 0)),
        out_shape=jax.ShapeDtypeStruct((G, C), jnp.float32),
        scratch_shapes=[
            pltpu.VMEM((G, H), jnp.float32),
            pltpu.VMEM((G, 1), jnp.float32),
        ],
    )


def kernel(x, edge_index, batch, Wl0, bl0, Wr0, Wl1, bl1, Wr1, Wl2, bl2, Wr2,
           W1, b1, W2, b2):
    N, H = x.shape
    C = W2.shape[1]
    E = edge_index.shape[1]
    K = 80
    assert E % (NW * K) == 0
    NCH = E // (NW * K)

    src = edge_index[0].reshape(NW, NCH * K)
    dst = edge_index[1].reshape(NW, NCH, K)

    agg = _make_agg(N, H, K, NCH)
    BE = 12800
    assert E % BE == 0
    NA = -(-((N + 127) // 128) // 8) * 8
    deghist = _make_deghist(E // BE, BE, NA)
    update0 = _make_update0(N, H, BN=5000)
    update = _make_update(N, H, BN=5000)
    update_pool = _make_update_pool(N, H, C, BN=5000)

    (part,) = agg(x, src, dst)
    dh = deghist(edge_index[1].reshape(E // BE, 1, BE))
    degcol = dh.reshape(-1)[:N].reshape(N, 1)
    h, rdeg = update0(part, degcol, x, Wl0, Wr0, bl0.reshape(1, H))
    (part,) = agg(h, src, dst)
    h = update(part, rdeg, h, Wl1, Wr1, bl1.reshape(1, H))
    (part,) = agg(h, src, dst)
    return update_pool(part, rdeg, h, Wl2, Wr2, bl2.reshape(1, H),
                       batch.reshape(N, 1), W1, b1.reshape(1, H), W2,
                       b2.reshape(1, C))
